# baseline (speedup 1.0000x reference)
import jax
import jax.numpy as jnp
from jax import lax
from jax.experimental import pallas as pl
from jax.experimental.pallas import tpu as pltpu
from jax.experimental.pallas import tpu_sc as plsc

ROWS = 100000
COLS = 512
N_TOTAL = float(ROWS * COLS)

SC_ROWS = 0
TC_ROWS = ROWS - SC_ROWS

TC_BLOCK_ROWS = 1000
TC_NUM_BLOCKS = TC_ROWS // TC_BLOCK_ROWS

NC = 2
NS = 16
NW = NC * NS
L = 16

SC_CHUNK_ROWS = 16
SC_GRID = SC_ROWS // SC_CHUNK_ROWS
SC_BLOCK_OFFSET = TC_ROWS // SC_CHUNK_ROWS
GROUPS_PER_ROW = COLS // L


def _tc_body(o_ref, t_ref, m_ref, out_ref, acc_ref):
    i = pl.program_id(0)

    d = o_ref[...] - t_ref[...]
    d2 = d * d
    m = m_ref[...].astype(jnp.float32)

    @pl.when(i == 0)
    def _init():
        acc_ref[0] = 0.0
        acc_ref[1] = 0.0
        acc_ref[2] = 0.0

    acc_ref[0] += jnp.sum(d2 * m)
    acc_ref[1] += jnp.sum(d2)
    acc_ref[2] += jnp.sum(m)

    @pl.when(i == TC_NUM_BLOCKS - 1)
    def _final():
        out_ref[0, 0] = acc_ref[0]
        out_ref[0, 1] = acc_ref[1]
        out_ref[0, 2] = acc_ref[2]


def _tc_partials(output, target, mask):
    return pl.pallas_call(
        _tc_body,
        grid=(TC_NUM_BLOCKS,),
        in_specs=[
            pl.BlockSpec((TC_BLOCK_ROWS, COLS), lambda i: (i, 0)),
            pl.BlockSpec((TC_BLOCK_ROWS, COLS), lambda i: (i, 0)),
            pl.BlockSpec((TC_BLOCK_ROWS, COLS), lambda i: (i, 0)),
        ],
        out_specs=pl.BlockSpec(
            (1, 3), lambda i: (0, 0), memory_space=pltpu.SMEM
        ),
        out_shape=jax.ShapeDtypeStruct((1, 3), jnp.float32),
        scratch_shapes=[pltpu.SMEM((3,), jnp.float32)],
    )(output, target, mask)


def _sc_block_body(acc_ref, o_v, t_v, m_v):
    def row_step(r, carry):
        sm0, st0, c0, sm1, st1, c1 = carry
        for g in range(GROUPS_PER_ROW):
            sl = (r, pl.ds(g * L, L))
            o = o_v[sl]
            t = t_v[sl]
            mf = m_v[sl].astype(jnp.float32)
            d = o - t
            d2 = d * d
            if g % 2 == 0:
                sm0 = sm0 + d2 * mf
                st0 = st0 + d2
                c0 = c0 + mf
            else:
                sm1 = sm1 + d2 * mf
                st1 = st1 + d2
                c1 = c1 + mf
        return sm0, st0, c0, sm1, st1, c1

    z = jnp.zeros((L,), jnp.float32)
    sm0, st0, c0, sm1, st1, c1 = lax.fori_loop(
        0, SC_CHUNK_ROWS, row_step, (z, z, z, z, z, z)
    )
    acc_ref[0, :] += sm0 + sm1
    acc_ref[1, :] += st0 + st1
    acc_ref[2, :] += c0 + c1


def _sc_partials(output, target, mask):
    mesh = plsc.VectorSubcoreMesh(core_axis_name="c", subcore_axis_name="s")

    @pl.kernel(
        out_type=jax.ShapeDtypeStruct((NW, 3, L), jnp.float32),
        mesh=mesh,
        scratch_types=[pltpu.VMEM((3, L), jnp.float32)],
    )
    def sc_kernel(o_hbm, t_hbm, m_hbm, out_hbm, acc_ref):
        wid = lax.axis_index("c") * NS + lax.axis_index("s")
        z = jnp.zeros((L,), jnp.float32)
        acc_ref[0, :] = z
        acc_ref[1, :] = z
        acc_ref[2, :] = z

        def body(o_v, t_v, m_v):
            _sc_block_body(acc_ref, o_v, t_v, m_v)

        pltpu.emit_pipeline(
            body,
            grid=(SC_GRID,),
            in_specs=[
                pl.BlockSpec(
                    (SC_CHUNK_ROWS, COLS),
                    index_map=lambda i: (SC_BLOCK_OFFSET + i, 0),
                ),
                pl.BlockSpec(
                    (SC_CHUNK_ROWS, COLS),
                    index_map=lambda i: (SC_BLOCK_OFFSET + i, 0),
                ),
                pl.BlockSpec(
                    (SC_CHUNK_ROWS, COLS),
                    index_map=lambda i: (SC_BLOCK_OFFSET + i, 0),
                ),
            ],
            out_specs=[],
            core_axis_name=("c", "s"),
            dimension_semantics=(pltpu.PARALLEL,),
        )(o_hbm, t_hbm, m_hbm)

        pltpu.sync_copy(acc_ref, out_hbm.at[wid])

    return sc_kernel(output, target, mask)


def kernel(output, target, mask):
    tc = _tc_partials(output, target, mask)
    if SC_ROWS:
        sc = _sc_partials(output, target, mask)
        s_m = tc[0, 0] + jnp.sum(sc[:, 0, :])
        s_tot = tc[0, 1] + jnp.sum(sc[:, 1, :])
        cnt = tc[0, 2] + jnp.sum(sc[:, 2, :])
    else:
        s_m, s_tot, cnt = tc[0, 0], tc[0, 1], tc[0, 2]

    return s_m / jnp.maximum(cnt, 1.0) + (s_tot - s_m) / jnp.maximum(
        N_TOTAL - cnt, 1.0
    )


# speedup vs baseline: 1.0634x; 1.0634x over previous
import jax
import jax.numpy as jnp
from jax import lax
from jax.experimental import pallas as pl
from jax.experimental.pallas import tpu as pltpu
from jax.experimental.pallas import tpu_sc as plsc

ROWS = 100000
COLS = 512
N_TOTAL = float(ROWS * COLS)

SC_ROWS = 0
TC_ROWS = ROWS - SC_ROWS

TC_BLOCK_ROWS = 2000
TC_NUM_BLOCKS = TC_ROWS // TC_BLOCK_ROWS

NC = 2
NS = 16
NW = NC * NS
L = 16

SC_CHUNK_ROWS = 16
SC_GRID = SC_ROWS // SC_CHUNK_ROWS
SC_BLOCK_OFFSET = TC_ROWS // SC_CHUNK_ROWS
GROUPS_PER_ROW = COLS // L


def _tc_body(o_ref, t_ref, m_ref, out_ref, acc_ref):
    i = pl.program_id(0)

    d = o_ref[...] - t_ref[...]
    d2 = d * d
    m = m_ref[...].astype(jnp.float32)

    @pl.when(i == 0)
    def _init():
        acc_ref[0] = 0.0
        acc_ref[1] = 0.0
        acc_ref[2] = 0.0

    acc_ref[0] += jnp.sum(d2 * m)
    acc_ref[1] += jnp.sum(d2)
    acc_ref[2] += jnp.sum(m)

    @pl.when(i == TC_NUM_BLOCKS - 1)
    def _final():
        out_ref[0, 0] = acc_ref[0]
        out_ref[0, 1] = acc_ref[1]
        out_ref[0, 2] = acc_ref[2]


def _tc_partials(output, target, mask):
    return pl.pallas_call(
        _tc_body,
        grid=(TC_NUM_BLOCKS,),
        in_specs=[
            pl.BlockSpec((TC_BLOCK_ROWS, COLS), lambda i: (i, 0)),
            pl.BlockSpec((TC_BLOCK_ROWS, COLS), lambda i: (i, 0)),
            pl.BlockSpec((TC_BLOCK_ROWS, COLS), lambda i: (i, 0)),
        ],
        out_specs=pl.BlockSpec(
            (1, 3), lambda i: (0, 0), memory_space=pltpu.SMEM
        ),
        out_shape=jax.ShapeDtypeStruct((1, 3), jnp.float32),
        scratch_shapes=[pltpu.SMEM((3,), jnp.float32)],
    )(output, target, mask)


def _sc_block_body(acc_ref, o_v, t_v, m_v):
    def row_step(r, carry):
        sm0, st0, c0, sm1, st1, c1 = carry
        for g in range(GROUPS_PER_ROW):
            sl = (r, pl.ds(g * L, L))
            o = o_v[sl]
            t = t_v[sl]
            mf = m_v[sl].astype(jnp.float32)
            d = o - t
            d2 = d * d
            if g % 2 == 0:
                sm0 = sm0 + d2 * mf
                st0 = st0 + d2
                c0 = c0 + mf
            else:
                sm1 = sm1 + d2 * mf
                st1 = st1 + d2
                c1 = c1 + mf
        return sm0, st0, c0, sm1, st1, c1

    z = jnp.zeros((L,), jnp.float32)
    sm0, st0, c0, sm1, st1, c1 = lax.fori_loop(
        0, SC_CHUNK_ROWS, row_step, (z, z, z, z, z, z)
    )
    acc_ref[0, :] += sm0 + sm1
    acc_ref[1, :] += st0 + st1
    acc_ref[2, :] += c0 + c1


def _sc_partials(output, target, mask):
    mesh = plsc.VectorSubcoreMesh(core_axis_name="c", subcore_axis_name="s")

    @pl.kernel(
        out_type=jax.ShapeDtypeStruct((NW, 3, L), jnp.float32),
        mesh=mesh,
        scratch_types=[pltpu.VMEM((3, L), jnp.float32)],
    )
    def sc_kernel(o_hbm, t_hbm, m_hbm, out_hbm, acc_ref):
        wid = lax.axis_index("c") * NS + lax.axis_index("s")
        z = jnp.zeros((L,), jnp.float32)
        acc_ref[0, :] = z
        acc_ref[1, :] = z
        acc_ref[2, :] = z

        def body(o_v, t_v, m_v):
            _sc_block_body(acc_ref, o_v, t_v, m_v)

        pltpu.emit_pipeline(
            body,
            grid=(SC_GRID,),
            in_specs=[
                pl.BlockSpec(
                    (SC_CHUNK_ROWS, COLS),
                    index_map=lambda i: (SC_BLOCK_OFFSET + i, 0),
                ),
                pl.BlockSpec(
                    (SC_CHUNK_ROWS, COLS),
                    index_map=lambda i: (SC_BLOCK_OFFSET + i, 0),
                ),
                pl.BlockSpec(
                    (SC_CHUNK_ROWS, COLS),
                    index_map=lambda i: (SC_BLOCK_OFFSET + i, 0),
                ),
            ],
            out_specs=[],
            core_axis_name=("c", "s"),
            dimension_semantics=(pltpu.PARALLEL,),
        )(o_hbm, t_hbm, m_hbm)

        pltpu.sync_copy(acc_ref, out_hbm.at[wid])

    return sc_kernel(output, target, mask)


def kernel(output, target, mask):
    tc = _tc_partials(output, target, mask)
    if SC_ROWS:
        sc = _sc_partials(output, target, mask)
        s_m = tc[0, 0] + jnp.sum(sc[:, 0, :])
        s_tot = tc[0, 1] + jnp.sum(sc[:, 1, :])
        cnt = tc[0, 2] + jnp.sum(sc[:, 2, :])
    else:
        s_m, s_tot, cnt = tc[0, 0], tc[0, 1], tc[0, 2]

    return s_m / jnp.maximum(cnt, 1.0) + (s_tot - s_m) / jnp.maximum(
        N_TOTAL - cnt, 1.0
    )


# speedup vs baseline: 1.0930x; 1.0279x over previous
import jax
import jax.numpy as jnp
from jax.experimental import pallas as pl
from jax.experimental.pallas import tpu as pltpu

ROWS = 100000
COLS = 512
BLOCK_ROWS = 2000
NUM_BLOCKS = ROWS // BLOCK_ROWS
N_TOTAL = float(ROWS * COLS)


def _body(o_ref, t_ref, m_ref, loss_ref, acc_ref):
    i = pl.program_id(0)

    d = o_ref[...] - t_ref[...]
    d2 = d * d
    m = m_ref[...].astype(jnp.float32)

    psum_m = jnp.sum(d2 * m)
    psum_tot = jnp.sum(d2)
    pcnt = jnp.sum(m)

    @pl.when(i == 0)
    def _init():
        acc_ref[0] = 0.0
        acc_ref[1] = 0.0
        acc_ref[2] = 0.0

    acc_ref[0] += psum_m
    acc_ref[1] += psum_tot
    acc_ref[2] += pcnt

    @pl.when(i == NUM_BLOCKS - 1)
    def _final():
        s_m = acc_ref[0]
        s_tot = acc_ref[1]
        c = acc_ref[2]
        loss = s_m / jnp.maximum(c, 1.0) + (s_tot - s_m) / jnp.maximum(
            N_TOTAL - c, 1.0
        )
        loss_ref[0, 0] = loss


def kernel(output, target, mask):
    loss = pl.pallas_call(
        _body,
        grid=(NUM_BLOCKS,),
        in_specs=[
            pl.BlockSpec((BLOCK_ROWS, COLS), lambda i: (i, 0)),
            pl.BlockSpec((BLOCK_ROWS, COLS), lambda i: (i, 0)),
            pl.BlockSpec((BLOCK_ROWS, COLS), lambda i: (i, 0)),
        ],
        out_specs=pl.BlockSpec(
            (1, 1), lambda i: (0, 0), memory_space=pltpu.SMEM
        ),
        out_shape=jax.ShapeDtypeStruct((1, 1), jnp.float32),
        scratch_shapes=[pltpu.SMEM((3,), jnp.float32)],
    )(output, target, mask)
    return loss[0, 0]
